# Initial kernel scaffold; baseline (speedup 1.0000x reference)
import jax
import jax.numpy as jnp
from jax.experimental import pallas as pl


def kernel(x, edge_index, num_nodes, params):
    raise NotImplementedError("write your pallas kernel here")


# speedup vs baseline: 3.4307x; 3.4307x over previous
import functools
import math

import jax
import jax.numpy as jnp
from jax import lax
from jax.experimental import pallas as pl
from jax.experimental.pallas import tpu as pltpu
from jax.experimental.pallas import tpu_sc as plsc

N_NODES = 50000
N_EDGES = 800000
H = 64

NC = 2
NS = 16
C = 128

NPAD = 50176
RPT = NPAD // NS
EPAD = 802816
EPT = EPAD // NS
NCHUNK = EPT // C

R = 512
GRID = NPAD // R

_mesh = plsc.VectorSubcoreMesh(core_axis_name="c", subcore_axis_name="s")


def _make_edge_kernel():
  out_type = [
      jax.ShapeDtypeStruct((NPAD, 32), jnp.float32),
      jax.ShapeDtypeStruct((NPAD, 32), jnp.float32),
  ]
  scratch = [
      pltpu.VMEM((C,), jnp.int32),
      pltpu.VMEM((C,), jnp.int32),
      pltpu.VMEM((C,), jnp.int32),
      pltpu.VMEM((C, 32), jnp.float32),
      pltpu.VMEM((C, 32), jnp.float32),
      pltpu.VMEM_SHARED((NPAD, 32), jnp.float32),
      pltpu.SemaphoreType.DMA,
  ]

  def body(dst_hbm, src_hbm, a_stk, b_stk, zeros_s,
           s0_out, s1_out,
           dstv, dstvo, srcv, ra, rb, s_sh, sem):
    c = lax.axis_index("c")
    s = lax.axis_index("s")
    coff = (c * NPAD).astype(jnp.int32)
    row0 = s * RPT

    pltpu.sync_copy(zeros_s, s_sh.at[pl.ds(row0, RPT)])
    plsc.subcore_barrier()

    def chunk_body(k, carry):
      off = s * EPT + k * C
      pltpu.sync_copy(dst_hbm.at[pl.ds(off, C)], dstv)
      pltpu.sync_copy(src_hbm.at[pl.ds(off, C)], srcv)
      for j in range(C // 16):
        sl = pl.ds(j * 16, 16)
        dstvo[sl] = dstv[sl] + coff
        srcv[sl] = srcv[sl] + coff
      ga = pltpu.async_copy(a_stk.at[dstvo], ra, sem)
      gb = pltpu.async_copy(b_stk.at[srcv], rb, sem)
      ga.wait()
      gb.wait()

      def row_body(r, carry2):
        s0 = pl.ds(0, 16)
        s1 = pl.ds(16, 16)
        ra[r, s0] = jnp.maximum(ra[r, s0] + rb[r, s0], 0.0)
        ra[r, s1] = jnp.maximum(ra[r, s1] + rb[r, s1], 0.0)
        return carry2

      lax.fori_loop(0, C, row_body, 0)
      pltpu.sync_copy(ra, s_sh.at[dstv], add=True)
      return carry

    lax.fori_loop(0, NCHUNK, chunk_body, 0)
    plsc.subcore_barrier()

    @pl.when(c == 0)
    def _():
      pltpu.sync_copy(s_sh.at[pl.ds(row0, RPT)], s0_out.at[pl.ds(row0, RPT)])

    @pl.when(c == 1)
    def _():
      pltpu.sync_copy(s_sh.at[pl.ds(row0, RPT)], s1_out.at[pl.ds(row0, RPT)])

  return pl.kernel(
      body, mesh=_mesh, out_type=out_type, scratch_types=scratch,
      compiler_params=pltpu.CompilerParams(use_tc_tiling_on_sc=False))


EPT2 = EPAD // (NC * NS)
NCHUNK2 = EPT2 // C


def _make_deg_kernel():
  out_type = [
      jax.ShapeDtypeStruct((NPAD, 8), jnp.float32),
      jax.ShapeDtypeStruct((NPAD, 8), jnp.float32),
  ]
  scratch = [
      pltpu.VMEM((C,), jnp.int32),
      pltpu.VMEM((C, 8), jnp.float32),
      pltpu.VMEM_SHARED((NPAD, 8), jnp.float32),
  ]

  def body(dst_hbm, zeros_d, ones_d, d0_out, d1_out, dstv, onesv, deg_sh):
    c = lax.axis_index("c")
    s = lax.axis_index("s")
    row0 = s * RPT
    pltpu.sync_copy(zeros_d, deg_sh.at[pl.ds(row0, RPT)])
    pltpu.sync_copy(ones_d, onesv)
    plsc.subcore_barrier()

    def chunk_body(k, carry):
      off = (c * NS + s) * EPT2 + k * C
      pltpu.sync_copy(dst_hbm.at[pl.ds(off, C)], dstv)
      pltpu.sync_copy(onesv, deg_sh.at[dstv], add=True)
      return carry

    lax.fori_loop(0, NCHUNK2, chunk_body, 0)
    plsc.subcore_barrier()

    @pl.when(c == 0)
    def _():
      pltpu.sync_copy(deg_sh.at[pl.ds(row0, RPT)], d0_out.at[pl.ds(row0, RPT)])

    @pl.when(c == 1)
    def _():
      pltpu.sync_copy(deg_sh.at[pl.ds(row0, RPT)], d1_out.at[pl.ds(row0, RPT)])

  return pl.kernel(
      body, mesh=_mesh, out_type=out_type, scratch_types=scratch,
      compiler_params=pltpu.CompilerParams(use_tc_tiling_on_sc=False))


_edge_k = _make_edge_kernel()
_deg_k = _make_deg_kernel()


def _dot(a, b):
  return jnp.dot(a, b, preferred_element_type=jnp.float32)


def _blk(shape):
  return pl.BlockSpec(shape, lambda i: (0,) * len(shape))


def _rowblk(cols):
  return pl.BlockSpec((R, cols), lambda i: (i, 0))


def _enc_body(x_ref, w1, b1, w2, b2, wi, bi, h_ref, ie_ref):
  x = x_ref[...]
  t = jnp.maximum(_dot(x, w1[...]) + b1[...], 0.0)
  t = jnp.maximum(_dot(t, w2[...]) + b2[...], 0.0)
  ie = _dot(x, wi[...]) + bi[...]
  h_ref[...] = t + ie
  ie_ref[...] = ie


def _encoder(xp, p):
  return pl.pallas_call(
      _enc_body,
      grid=(GRID,),
      in_specs=[_rowblk(2), _blk((2, 32)), _blk((1, 32)), _blk((32, H)),
                _blk((1, H)), _blk((2, H)), _blk((1, H))],
      out_specs=[_rowblk(H), _rowblk(H)],
      out_shape=[jax.ShapeDtypeStruct((NPAD, H), jnp.float32)] * 2,
  )(xp, p['enc_w1'], p['enc_b1'].reshape(1, -1), p['enc_w2'],
    p['enc_b2'].reshape(1, -1), p['inp_w'], p['inp_b'].reshape(1, -1))


def _make_pre_body(has_cat):
  def body(h_ref, ie_ref, cw1a, cw1b, cb1, cw2, cb2, mw1a, mw1b, mb1,
           hm_ref, a3_ref, b3_ref):
    h = h_ref[...]
    if has_cat:
      cc = jnp.maximum(_dot(h, cw1a[...]) + _dot(ie_ref[...], cw1b[...])
                       + cb1[...], 0.0)
      cc = _dot(cc, cw2[...]) + cb2[...]
      h = jnp.maximum(cc, 0.0)
    hm_ref[...] = h
    a = _dot(h, mw1a[...]) + mb1[...]
    b = _dot(h, mw1b[...])
    a3_ref[0] = a[:, :32]
    a3_ref[1] = a[:, 32:]
    b3_ref[0] = b[:, :32]
    b3_ref[1] = b[:, 32:]
  return body


def _pre(h, ie, p, i, has_cat):
  out = pl.pallas_call(
      _make_pre_body(has_cat),
      grid=(GRID,),
      in_specs=[_rowblk(H), _rowblk(H),
                _blk((H, H)), _blk((H, H)), _blk((1, H)),
                _blk((H, H)), _blk((1, H)),
                _blk((H, H)), _blk((H, H)), _blk((1, H))],
      out_specs=[_rowblk(H),
                 pl.BlockSpec((2, R, 32), lambda i: (0, i, 0)),
                 pl.BlockSpec((2, R, 32), lambda i: (0, i, 0))],
      out_shape=[jax.ShapeDtypeStruct((NPAD, H), jnp.float32),
                 jax.ShapeDtypeStruct((2, NPAD, 32), jnp.float32),
                 jax.ShapeDtypeStruct((2, NPAD, 32), jnp.float32)],
  )(h, ie, p['cat_w1'][:H], p['cat_w1'][H:], p['cat_b1'].reshape(1, -1),
    p['cat_w2'], p['cat_b2'].reshape(1, -1),
    p['msg_w1_%d' % i][:H], p['msg_w1_%d' % i][H:],
    p['msg_b1_%d' % i].reshape(1, -1))
  return out


def _post_body(hm_ref, s0_ref, s1_ref, dg0_ref, dg1_ref, w2a, w2b, mb2,
               uw1a, uw1b, ub1, uw2, ub2, out_ref):
  hm = hm_ref[...]
  dg = dg0_ref[...][:, 0:1] + dg1_ref[...][:, 0:1]
  agg = (_dot(s0_ref[...], w2a[...]) + _dot(s1_ref[...], w2b[...])
         + dg * mb2[...])
  u = jnp.maximum(_dot(hm, uw1a[...]) + _dot(agg, uw1b[...]) + ub1[...], 0.0)
  u = _dot(u, uw2[...]) + ub2[...]
  out_ref[...] = jnp.maximum(u + hm, 0.0)


def _post(hm, s0, s1, dg0, dg1, p, i):
  return pl.pallas_call(
      _post_body,
      grid=(GRID,),
      in_specs=[_rowblk(H), _rowblk(32), _rowblk(32), _rowblk(8), _rowblk(8),
                _blk((32, H)), _blk((32, H)), _blk((1, H)),
                _blk((H, H)), _blk((H, H)), _blk((1, H)),
                _blk((H, H)), _blk((1, H))],
      out_specs=_rowblk(H),
      out_shape=jax.ShapeDtypeStruct((NPAD, H), jnp.float32),
  )(hm, s0, s1, dg0, dg1,
    p['msg_w2_%d' % i][:32], p['msg_w2_%d' % i][32:],
    p['msg_b2_%d' % i].reshape(1, -1),
    p['upd_w1_%d' % i][:H], p['upd_w1_%d' % i][H:],
    p['upd_b1_%d' % i].reshape(1, -1),
    p['upd_w2_%d' % i], p['upd_b2_%d' % i].reshape(1, -1))


def _dec_body(h_ref, w1, b1, w2, b2, o_ref):
  d = jnp.maximum(_dot(h_ref[...], w1[...]) + b1[...], 0.0)
  z = _dot(d, w2[...]) + b2[...]
  m = jnp.max(z, axis=1, keepdims=True)
  e = jnp.exp(z - m)
  o_ref[...] = (z - m) - jnp.log(jnp.sum(e, axis=1, keepdims=True))


def _decoder(h, p):
  return pl.pallas_call(
      _dec_body,
      grid=(GRID,),
      in_specs=[_rowblk(H), _blk((H, 64)), _blk((1, 64)), _blk((64, 2)),
                _blk((1, 2))],
      out_specs=_rowblk(2),
      out_shape=jax.ShapeDtypeStruct((NPAD, 2), jnp.float32),
  )(h, p['dec_w1'], p['dec_b1'].reshape(1, -1), p['dec_w2'],
    p['dec_b2'].reshape(1, -1))


def kernel(x, edge_index, num_nodes, params):
  n = x.shape[0]
  e = edge_index.shape[1]
  p = params

  xp = jnp.pad(x, ((0, NPAD - n), (0, 0)))
  pad_e = EPAD - e
  fill = jnp.full((pad_e,), n, jnp.int32)
  src_p = jnp.concatenate([edge_index[0], fill])
  dst_p = jnp.concatenate([edge_index[1], fill])

  zeros_s = jnp.zeros((RPT, 32), jnp.float32)
  zeros_d = jnp.zeros((RPT, 8), jnp.float32)
  ones_d = jnp.ones((C, 8), jnp.float32)

  h, ie = _encoder(xp, p)
  dg0, dg1 = _deg_k(dst_p, zeros_d, ones_d)

  eff = min(8, max(4, int(math.log2(n))))
  for i in range(eff):
    hm, a3, b3 = _pre(h, ie, p, i, has_cat=(i > 0))
    a_stk = a3.reshape(2 * NPAD, 32)
    b_stk = b3.reshape(2 * NPAD, 32)
    s0, s1 = _edge_k(dst_p, src_p, a_stk, b_stk, zeros_s)
    h = _post(hm, s0, s1, dg0, dg1, p, i)

  out = _decoder(h, p)
  return out[:n]
